# Initial kernel scaffold; baseline (speedup 1.0000x reference)
import jax
import jax.numpy as jnp
from jax.experimental import pallas as pl


def kernel(x, edge_index, W_lin, W_l, b_l, W_r, gamma, beta):
    raise NotImplementedError("write your pallas kernel here")


# speedup vs baseline: 2.5233x; 2.5233x over previous
import jax
import jax.numpy as jnp
from jax import lax
from jax.experimental import pallas as pl
from jax.experimental.pallas import tpu as pltpu
from jax.experimental.pallas import tpu_sc as plsc

N_NODES = 10000
N_EDGES = 320000
D = 128

NC = 2
NS = 16
NW = NC * NS
CH = 64
EPT = 10240
NCHUNK = EPT // CH
E_PAD = NW * EPT
ROWS_PAD = 10240
RPT = ROWS_PAD // NS
RPC = RPT // CH

TC_BLK = 400


def _tc1_body(x_ref, wlt_ref, wllt_ref, wrt_ref, g_ref, r_ref):
    h = jnp.dot(x_ref[...], wlt_ref[...], preferred_element_type=jnp.float32)
    g_ref[...] = jnp.dot(h, wllt_ref[...], preferred_element_type=jnp.float32)
    r_ref[...] = jnp.dot(h, wrt_ref[...], preferred_element_type=jnp.float32)


def _tc1(x, wlt, wllt, wrt):
    grid = N_NODES // TC_BLK
    return pl.pallas_call(
        _tc1_body,
        grid=(grid,),
        in_specs=[
            pl.BlockSpec((TC_BLK, D), lambda i: (i, 0)),
            pl.BlockSpec((D, D), lambda i: (0, 0)),
            pl.BlockSpec((D, D), lambda i: (0, 0)),
            pl.BlockSpec((D, D), lambda i: (0, 0)),
        ],
        out_specs=[
            pl.BlockSpec((TC_BLK, D), lambda i: (i, 0)),
            pl.BlockSpec((TC_BLK, D), lambda i: (i, 0)),
        ],
        out_shape=[
            jax.ShapeDtypeStruct((N_NODES, D), jnp.float32),
            jax.ShapeDtypeStruct((N_NODES, D), jnp.float32),
        ],
    )(x, wlt, wllt, wrt)


def _sc_body(g_hbm, src_hbm, dst_hbm, zrow_hbm, ones_hbm,
             acc_out, cnt_out,
             sidx_v, didx_v, rows_v, ones_v, acc_sh, sem):
    c = lax.axis_index("c")
    s = lax.axis_index("s")

    pltpu.sync_copy(zrow_hbm, rows_v)
    pltpu.sync_copy(ones_hbm, ones_v)

    def initacc(k, carry):
        rb = pl.multiple_of(s * RPT + k * CH, CH)
        pltpu.sync_copy(rows_v, acc_sh.at[pl.ds(rb, CH)])
        return carry

    lax.fori_loop(0, RPC, initacc, 0)
    plsc.subcore_barrier()

    ebase = (c * NS + s) * EPT

    def feat(j, carry):
        off = pl.multiple_of(ebase + j * CH, CH)
        pltpu.sync_copy(src_hbm.at[pl.ds(off, CH)], sidx_v)
        pltpu.sync_copy(dst_hbm.at[pl.ds(off, CH)], didx_v)
        pltpu.async_copy(g_hbm.at[sidx_v], rows_v, sem).wait()
        pltpu.sync_copy(rows_v, acc_sh.at[didx_v], add=True)
        return carry

    lax.fori_loop(0, NCHUNK, feat, 0)
    plsc.subcore_barrier()

    for k in range(RPC):
        rb = pl.multiple_of(s * RPT + k * CH, CH)
        pltpu.sync_copy(acc_sh.at[pl.ds(rb, CH)], rows_v)
        pltpu.sync_copy(rows_v, acc_out.at[c, pl.ds(rb, CH)])

    pltpu.sync_copy(zrow_hbm, rows_v)
    lax.fori_loop(0, RPC, initacc, 0)
    plsc.subcore_barrier()

    def hist(j, carry):
        off = pl.multiple_of(ebase + j * CH, CH)
        pltpu.sync_copy(dst_hbm.at[pl.ds(off, CH)], didx_v)
        pltpu.sync_copy(ones_v, acc_sh.at[didx_v], add=True)
        return carry

    lax.fori_loop(0, NCHUNK, hist, 0)
    plsc.subcore_barrier()

    for k in range(RPC):
        rb = pl.multiple_of(s * RPT + k * CH, CH)
        pltpu.sync_copy(acc_sh.at[pl.ds(rb, CH)], rows_v)
        pltpu.sync_copy(rows_v, cnt_out.at[c, pl.ds(rb, CH)])


_sc_agg = pl.kernel(
    _sc_body,
    out_type=[
        jax.ShapeDtypeStruct((NC, ROWS_PAD, D), jnp.float32),
        jax.ShapeDtypeStruct((NC, ROWS_PAD, D), jnp.float32),
    ],
    mesh=plsc.VectorSubcoreMesh(core_axis_name="c", subcore_axis_name="s"),
    scratch_types=[
        pltpu.VMEM((CH,), jnp.int32),
        pltpu.VMEM((CH,), jnp.int32),
        pltpu.VMEM((CH, D), jnp.float32),
        pltpu.VMEM((CH, D), jnp.float32),
        pltpu.VMEM_SHARED((ROWS_PAD, D), jnp.float32),
        pltpu.SemaphoreType.DMA,
    ],
)


def _tc2_body(a0_ref, a1_ref, c0_ref, c1_ref, r_ref, bl_ref, gam_ref,
              bet_ref, o_ref):
    ssum = a0_ref[...] + a1_ref[...]
    cnt = c0_ref[:, :1] + c1_ref[:, :1]
    u = ssum / jnp.maximum(cnt, 1.0) + bl_ref[...] + r_ref[...]
    mu = jnp.mean(u, axis=-1, keepdims=True)
    d = u - mu
    var = jnp.mean(d * d, axis=-1, keepdims=True)
    ln = d * lax.rsqrt(var + 1e-5) * gam_ref[...] + bet_ref[...]
    o_ref[...] = jnp.maximum(ln, 0.0)


def _tc2(a0, a1, c0, c1, r, bl, gam, bet):
    grid = N_NODES // TC_BLK
    return pl.pallas_call(
        _tc2_body,
        grid=(grid,),
        in_specs=[
            pl.BlockSpec((TC_BLK, D), lambda i: (i, 0)),
            pl.BlockSpec((TC_BLK, D), lambda i: (i, 0)),
            pl.BlockSpec((TC_BLK, D), lambda i: (i, 0)),
            pl.BlockSpec((TC_BLK, D), lambda i: (i, 0)),
            pl.BlockSpec((TC_BLK, D), lambda i: (i, 0)),
            pl.BlockSpec((1, D), lambda i: (0, 0)),
            pl.BlockSpec((1, D), lambda i: (0, 0)),
            pl.BlockSpec((1, D), lambda i: (0, 0)),
        ],
        out_specs=pl.BlockSpec((TC_BLK, D), lambda i: (i, 0)),
        out_shape=jax.ShapeDtypeStruct((N_NODES, D), jnp.float32),
    )(a0, a1, c0, c1, r, bl, gam, bet)


def kernel(x, edge_index, W_lin, W_l, b_l, W_r, gamma, beta):
    ei = edge_index.astype(jnp.int32)
    pad = E_PAD - N_EDGES
    src = jnp.concatenate([ei[0], jnp.zeros((pad,), jnp.int32)])
    dst = jnp.concatenate([ei[1], jnp.full((pad,), N_NODES, jnp.int32)])

    g, r = _tc1(x, W_lin.T, W_l.T, W_r.T)

    zrow = jnp.zeros((CH, D), jnp.float32)
    ones = jnp.ones((CH, D), jnp.float32)
    acc, cnt = _sc_agg(g, src, dst, zrow, ones)

    return _tc2(acc[0], acc[1], cnt[0], cnt[1], r,
                b_l.reshape(1, D), gamma.reshape(1, D), beta.reshape(1, D))


# speedup vs baseline: 2.6965x; 1.0686x over previous
import jax
import jax.numpy as jnp
from jax import lax
from jax.experimental import pallas as pl
from jax.experimental.pallas import tpu as pltpu
from jax.experimental.pallas import tpu_sc as plsc

N_NODES = 10000
N_EDGES = 320000
D = 128

NC = 2
NS = 16
NW = NC * NS
CH = 80
EPT = 10240
NCHUNK = EPT // CH
E_PAD = NW * EPT
ROWS_PAD = 10240
RPT = ROWS_PAD // NS
RPC = RPT // CH

TC_BLK = 400


def _tc1_body(x_ref, wlt_ref, wllt_ref, wrt_ref, g_ref, r_ref):
    h = jnp.dot(x_ref[...], wlt_ref[...], preferred_element_type=jnp.float32)
    g_ref[...] = jnp.dot(h, wllt_ref[...], preferred_element_type=jnp.float32)
    r_ref[...] = jnp.dot(h, wrt_ref[...], preferred_element_type=jnp.float32)


def _tc1(x, wlt, wllt, wrt):
    grid = N_NODES // TC_BLK
    return pl.pallas_call(
        _tc1_body,
        grid=(grid,),
        in_specs=[
            pl.BlockSpec((TC_BLK, D), lambda i: (i, 0)),
            pl.BlockSpec((D, D), lambda i: (0, 0)),
            pl.BlockSpec((D, D), lambda i: (0, 0)),
            pl.BlockSpec((D, D), lambda i: (0, 0)),
        ],
        out_specs=[
            pl.BlockSpec((TC_BLK, D), lambda i: (i, 0)),
            pl.BlockSpec((TC_BLK, D), lambda i: (i, 0)),
        ],
        out_shape=[
            jax.ShapeDtypeStruct((N_NODES, D), jnp.float32),
            jax.ShapeDtypeStruct((N_NODES, D), jnp.float32),
        ],
    )(x, wlt, wllt, wrt)


def _sc_body(g_hbm, src_hbm, dst_hbm, zrow_hbm, ones_hbm,
             acc_out, cnt_out,
             sidx_v, didx_v, rows_v, ones_v, acc_sh, sem):
    c = lax.axis_index("c")
    s = lax.axis_index("s")

    pltpu.sync_copy(zrow_hbm, rows_v)
    pltpu.sync_copy(ones_hbm, ones_v)

    def initacc(k, carry):
        rb = pl.multiple_of(s * RPT + k * CH, CH)
        pltpu.sync_copy(rows_v, acc_sh.at[pl.ds(rb, CH)])
        return carry

    lax.fori_loop(0, RPC, initacc, 0)
    plsc.subcore_barrier()

    ebase = (c * NS + s) * EPT

    def feat(j, carry):
        off = pl.multiple_of(ebase + j * CH, CH)
        pltpu.sync_copy(src_hbm.at[pl.ds(off, CH)], sidx_v)
        pltpu.sync_copy(dst_hbm.at[pl.ds(off, CH)], didx_v)
        pltpu.async_copy(g_hbm.at[sidx_v], rows_v, sem).wait()
        pltpu.sync_copy(rows_v, acc_sh.at[didx_v], add=True)
        return carry

    lax.fori_loop(0, NCHUNK, feat, 0)
    plsc.subcore_barrier()

    for k in range(RPC):
        rb = pl.multiple_of(s * RPT + k * CH, CH)
        pltpu.sync_copy(acc_sh.at[pl.ds(rb, CH)], rows_v)
        pltpu.sync_copy(rows_v, acc_out.at[c, pl.ds(rb, CH)])

    pltpu.sync_copy(zrow_hbm, rows_v)
    lax.fori_loop(0, RPC, initacc, 0)
    plsc.subcore_barrier()

    def hist(j, carry):
        off = pl.multiple_of(ebase + j * CH, CH)
        pltpu.sync_copy(dst_hbm.at[pl.ds(off, CH)], didx_v)
        pltpu.sync_copy(ones_v, acc_sh.at[didx_v], add=True)
        return carry

    lax.fori_loop(0, NCHUNK, hist, 0)
    plsc.subcore_barrier()

    for k in range(RPC):
        rb = pl.multiple_of(s * RPT + k * CH, CH)
        pltpu.sync_copy(acc_sh.at[pl.ds(rb, CH)], rows_v)
        pltpu.sync_copy(rows_v, cnt_out.at[c, pl.ds(rb, CH)])


_sc_agg = pl.kernel(
    _sc_body,
    out_type=[
        jax.ShapeDtypeStruct((NC, ROWS_PAD, D), jnp.float32),
        jax.ShapeDtypeStruct((NC, ROWS_PAD, D), jnp.float32),
    ],
    mesh=plsc.VectorSubcoreMesh(core_axis_name="c", subcore_axis_name="s"),
    scratch_types=[
        pltpu.VMEM((CH,), jnp.int32),
        pltpu.VMEM((CH,), jnp.int32),
        pltpu.VMEM((CH, D), jnp.float32),
        pltpu.VMEM((CH, D), jnp.float32),
        pltpu.VMEM_SHARED((ROWS_PAD, D), jnp.float32),
        pltpu.SemaphoreType.DMA,
    ],
)


def _tc2_body(a0_ref, a1_ref, c0_ref, c1_ref, r_ref, bl_ref, gam_ref,
              bet_ref, o_ref):
    ssum = a0_ref[...] + a1_ref[...]
    cnt = c0_ref[:, :1] + c1_ref[:, :1]
    u = ssum / jnp.maximum(cnt, 1.0) + bl_ref[...] + r_ref[...]
    mu = jnp.mean(u, axis=-1, keepdims=True)
    d = u - mu
    var = jnp.mean(d * d, axis=-1, keepdims=True)
    ln = d * lax.rsqrt(var + 1e-5) * gam_ref[...] + bet_ref[...]
    o_ref[...] = jnp.maximum(ln, 0.0)


def _tc2(a0, a1, c0, c1, r, bl, gam, bet):
    grid = N_NODES // TC_BLK
    return pl.pallas_call(
        _tc2_body,
        grid=(grid,),
        in_specs=[
            pl.BlockSpec((TC_BLK, D), lambda i: (i, 0)),
            pl.BlockSpec((TC_BLK, D), lambda i: (i, 0)),
            pl.BlockSpec((TC_BLK, D), lambda i: (i, 0)),
            pl.BlockSpec((TC_BLK, D), lambda i: (i, 0)),
            pl.BlockSpec((TC_BLK, D), lambda i: (i, 0)),
            pl.BlockSpec((1, D), lambda i: (0, 0)),
            pl.BlockSpec((1, D), lambda i: (0, 0)),
            pl.BlockSpec((1, D), lambda i: (0, 0)),
        ],
        out_specs=pl.BlockSpec((TC_BLK, D), lambda i: (i, 0)),
        out_shape=jax.ShapeDtypeStruct((N_NODES, D), jnp.float32),
    )(a0, a1, c0, c1, r, bl, gam, bet)


def kernel(x, edge_index, W_lin, W_l, b_l, W_r, gamma, beta):
    ei = edge_index.astype(jnp.int32)
    pad = E_PAD - N_EDGES
    src = jnp.concatenate([ei[0], jnp.zeros((pad,), jnp.int32)])
    dst = jnp.concatenate([ei[1], jnp.full((pad,), N_NODES, jnp.int32)])

    g, r = _tc1(x, W_lin.T, W_l.T, W_r.T)

    zrow = jnp.zeros((CH, D), jnp.float32)
    ones = jnp.ones((CH, D), jnp.float32)
    acc, cnt = _sc_agg(g, src, dst, zrow, ones)

    return _tc2(acc[0], acc[1], cnt[0], cnt[1], r,
                b_l.reshape(1, D), gamma.reshape(1, D), beta.reshape(1, D))
